# baseline (speedup 1.0000x reference)
import functools

import jax
import jax.numpy as jnp
from jax import lax
from jax.experimental import pallas as pl
from jax.experimental.pallas import tpu as pltpu
from jax.experimental.pallas import tpu_sc as plsc

NC = 2
NS = 16
NW = NC * NS
CHUNK = 128
K = 5
LANES = 16


@functools.cache
def _build_transpose(V, D):
    n_full = V // 128
    rem = V - n_full * 128
    vpad = (n_full + (1 if rem else 0)) * 128
    mesh = plsc.VectorSubcoreMesh(core_axis_name="c", subcore_axis_name="s")

    @functools.partial(
        pl.kernel,
        out_type=jax.ShapeDtypeStruct((vpad // 2, 128), jnp.float32),
        mesh=mesh,
        scratch_types=[
            pltpu.VMEM((D, 128), jnp.float32),
            pltpu.VMEM((64, 128), jnp.float32),
        ],
        compiler_params=pltpu.CompilerParams(needs_layout_passes=False),
    )
    def tr_kernel(wt_hbm, wtail_hbm, wd_hbm, tbuf, obuf):
        wid = lax.axis_index("s") * NC + lax.axis_index("c")
        iotas = [
            jnp.arange(16 * k4, 16 * (k4 + 1), dtype=jnp.int32)
            for k4 in range(D // LANES)
        ]

        def do_tile(t):
            pltpu.sync_copy(wt_hbm.at[:, pl.ds(128 * t, 128)], tbuf)

            @pl.loop(0, 64, unroll=8)
            def _(p):
                for h in range(2):
                    col = jnp.full((16,), 2 * p + h, dtype=jnp.int32)
                    for k4 in range(D // LANES):
                        v = plsc.load_gather(tbuf, [iotas[k4], col])
                        obuf[p, pl.ds(64 * h + LANES * k4, LANES)] = v

            pltpu.sync_copy(obuf, wd_hbm.at[pl.ds(64 * t, 64), :])

        @pl.loop(0, (n_full - 1) // NW + 1)
        def _(k):
            t = wid + NW * k

            @pl.when(t < n_full)
            def _():
                do_tile(t)

        if rem:
            @pl.when(wid == NW - 1)
            def _():
                pltpu.sync_copy(wtail_hbm, obuf.at[pl.ds(0, rem // 2), :])
                pltpu.sync_copy(obuf.at[pl.ds(0, rem // 2), :],
                                wd_hbm.at[pl.ds(64 * n_full, rem // 2), :])

    return tr_kernel, vpad, rem


@functools.cache
def _build_gather(n_chunks, D):
    n_groups = n_chunks // K
    assert n_groups % 2 == 0 and n_groups >= 4
    mesh = plsc.VectorSubcoreMesh(core_axis_name="c", subcore_axis_name="s")

    @functools.partial(
        pl.kernel,
        out_type=jax.ShapeDtypeStruct((NW, n_chunks, CHUNK, D), jnp.float32),
        mesh=mesh,
        scratch_types=[
            pltpu.VMEM((n_chunks, CHUNK), jnp.int32),
            pltpu.VMEM((2, K, CHUNK, D), jnp.float32),
            pltpu.SemaphoreType.DMA,
            pltpu.SemaphoreType.DMA,
            pltpu.SemaphoreType.DMA,
            pltpu.SemaphoreType.DMA,
        ],
        compiler_params=pltpu.CompilerParams(use_tc_tiling_on_sc=False),
    )
    def emb_kernel(idx_hbm, table_hbm, out_hbm, idx_v, rows_v,
                   gsem_a, gsem_b, ssem_a, ssem_b):
        wid = lax.axis_index("s") * NC + lax.axis_index("c")
        pltpu.sync_copy(idx_hbm.at[wid], idx_v)

        def fire_gathers(g, half, sem):
            for b in range(K):
                pltpu.async_copy(
                    table_hbm.at[idx_v.at[g * K + b]], rows_v.at[half, b], sem)

        def drain_gathers(half, sem):
            for b in range(K):
                pltpu.make_async_copy(
                    table_hbm.at[idx_v.at[0]], rows_v.at[half, b], sem).wait()

        def fire_stores(g, half, sem):
            for b in range(K):
                pltpu.async_copy(
                    rows_v.at[half, b], out_hbm.at[wid, g * K + b], sem)

        def drain_stores(half, sem):
            for b in range(K):
                pltpu.make_async_copy(
                    rows_v.at[half, b], out_hbm.at[wid, b], sem).wait()

        fire_gathers(0, 0, gsem_a)
        drain_gathers(0, gsem_a)
        fire_gathers(1, 1, gsem_b)
        fire_stores(0, 0, ssem_a)

        @pl.loop(1, n_groups - 1, step=2)
        def _(g):
            drain_gathers(1, gsem_b)
            drain_stores(0, ssem_a)
            fire_gathers(g + 1, 0, gsem_a)
            fire_stores(g, 1, ssem_b)

            drain_gathers(0, gsem_a)
            drain_stores(1, ssem_b)
            fire_gathers(g + 2, 1, gsem_b)
            fire_stores(g + 1, 0, ssem_a)

        drain_gathers(1, gsem_b)
        drain_stores(0, ssem_a)
        fire_stores(n_groups - 1, 1, ssem_b)
        drain_stores(1, ssem_b)

    return emb_kernel


def kernel(x, W):
    B, H = x.shape
    V, D = W.shape
    n_flat = B * H
    assert n_flat % (NW * CHUNK) == 0 and D == 64
    n_chunks = n_flat // (NW * CHUNK)

    tr, vpad, rem = _build_transpose(V, D)
    assert rem > 0 and rem % 2 == 0
    wtail = W[V - rem:].reshape(rem // 2, 2 * D)
    Wd = tr(W.T, wtail)
    table = Wd.reshape(vpad, D)

    idx = x.reshape(NW, n_chunks, CHUNK)
    out = _build_gather(n_chunks, D)(idx, table)
    return out.reshape(B, H * D)


# speedup vs baseline: 1.4672x; 1.4672x over previous
import functools

import jax
import jax.numpy as jnp
from jax import lax
from jax.experimental import pallas as pl
from jax.experimental.pallas import tpu as pltpu
from jax.experimental.pallas import tpu_sc as plsc

NC = 2
NS = 16
NW = NC * NS
CHUNK = 128
K = 5
LANES = 16


@functools.cache
def _build_transpose(V, D):
    n_full = V // 128
    rem = V - n_full * 128
    vpad = (n_full + (1 if rem else 0)) * 128
    mesh = plsc.VectorSubcoreMesh(core_axis_name="c", subcore_axis_name="s")

    n_iter = (n_full - 1) // NW + 1
    if n_iter % 2:
        n_iter += 1

    @functools.partial(
        pl.kernel,
        out_type=jax.ShapeDtypeStruct((vpad // 2, 128), jnp.float32),
        mesh=mesh,
        scratch_types=[
            pltpu.VMEM((2, D, 128), jnp.float32),
            pltpu.VMEM((2, 64, 128), jnp.float32),
            pltpu.SemaphoreType.DMA,
            pltpu.SemaphoreType.DMA,
            pltpu.SemaphoreType.DMA,
            pltpu.SemaphoreType.DMA,
        ],
        compiler_params=pltpu.CompilerParams(needs_layout_passes=False),
    )
    def tr_kernel(wt_hbm, wtail_hbm, wd_hbm, tbuf, obuf,
                  isem_a, isem_b, osem_a, osem_b):
        wid = lax.axis_index("s") * NC + lax.axis_index("c")
        isems = (isem_a, isem_b)
        osems = (osem_a, osem_b)
        lane = jnp.arange(16, dtype=jnp.int32)
        rows = [lane // 2 + 8 * j for j in range(8)]
        colbase = (lane % 2) * 64

        def tile_of(k):
            return wid + NW * k

        def fire_in(k, par):
            @pl.when(tile_of(k) < n_full)
            def _():
                pltpu.async_copy(
                    wt_hbm.at[:, pl.ds(128 * tile_of(k), 128)],
                    tbuf.at[par], isems[par])

        def wait_in(par):
            pltpu.make_async_copy(
                wt_hbm.at[:, pl.ds(0, 128)], tbuf.at[par], isems[par]).wait()

        def fire_out(k, par):
            pltpu.async_copy(
                obuf.at[par], wd_hbm.at[pl.ds(64 * tile_of(k), 64), :],
                osems[par])

        def wait_out(par):
            pltpu.make_async_copy(
                obuf.at[par], wd_hbm.at[pl.ds(0, 64), :], osems[par]).wait()

        def compute(par):
            @pl.loop(0, D)
            def _(d):
                col = colbase + d
                for j in range(8):
                    v = tbuf[par, d, pl.ds(16 * j, 16)]
                    plsc.store_scatter(obuf.at[par], [rows[j], col], v)

        def step(k, par):
            fire_in(k + 1, 1 - par)

            @pl.when(tile_of(k) < n_full)
            def _():
                wait_in(par)

            @pl.when(jnp.logical_and(k >= 2, tile_of(k - 2) < n_full))
            def _():
                wait_out(par)

            @pl.when(tile_of(k) < n_full)
            def _():
                compute(par)
                fire_out(k, par)

        fire_in(0, 0)

        @pl.loop(0, n_iter, step=2)
        def _(k):
            step(k, 0)
            step(k + 1, 1)

        for m in (n_iter - 2, n_iter - 1):
            @pl.when(tile_of(m) < n_full)
            def _():
                wait_out(m % 2)

        if rem:
            @pl.when(wid == NW - 1)
            def _():
                pltpu.sync_copy(wtail_hbm, obuf.at[0, pl.ds(0, rem // 2), :])
                pltpu.sync_copy(obuf.at[0, pl.ds(0, rem // 2), :],
                                wd_hbm.at[pl.ds(64 * n_full, rem // 2), :])

    return tr_kernel, vpad, rem


@functools.cache
def _build_gather(n_chunks, D):
    n_groups = n_chunks // K
    assert n_groups % 2 == 0 and n_groups >= 4
    mesh = plsc.VectorSubcoreMesh(core_axis_name="c", subcore_axis_name="s")

    @functools.partial(
        pl.kernel,
        out_type=jax.ShapeDtypeStruct((NW, n_chunks, CHUNK, D), jnp.float32),
        mesh=mesh,
        scratch_types=[
            pltpu.VMEM((n_chunks, CHUNK), jnp.int32),
            pltpu.VMEM((2, K, CHUNK, D), jnp.float32),
            pltpu.SemaphoreType.DMA,
            pltpu.SemaphoreType.DMA,
            pltpu.SemaphoreType.DMA,
            pltpu.SemaphoreType.DMA,
        ],
        compiler_params=pltpu.CompilerParams(use_tc_tiling_on_sc=False),
    )
    def emb_kernel(idx_hbm, table_hbm, out_hbm, idx_v, rows_v,
                   gsem_a, gsem_b, ssem_a, ssem_b):
        wid = lax.axis_index("s") * NC + lax.axis_index("c")
        pltpu.sync_copy(idx_hbm.at[wid], idx_v)

        def fire_gathers(g, half, sem):
            for b in range(K):
                pltpu.async_copy(
                    table_hbm.at[idx_v.at[g * K + b]], rows_v.at[half, b], sem)

        def drain_gathers(half, sem):
            for b in range(K):
                pltpu.make_async_copy(
                    table_hbm.at[idx_v.at[0]], rows_v.at[half, b], sem).wait()

        def fire_stores(g, half, sem):
            for b in range(K):
                pltpu.async_copy(
                    rows_v.at[half, b], out_hbm.at[wid, g * K + b], sem)

        def drain_stores(half, sem):
            for b in range(K):
                pltpu.make_async_copy(
                    rows_v.at[half, b], out_hbm.at[wid, b], sem).wait()

        fire_gathers(0, 0, gsem_a)
        drain_gathers(0, gsem_a)
        fire_gathers(1, 1, gsem_b)
        fire_stores(0, 0, ssem_a)

        @pl.loop(1, n_groups - 1, step=2)
        def _(g):
            drain_gathers(1, gsem_b)
            drain_stores(0, ssem_a)
            fire_gathers(g + 1, 0, gsem_a)
            fire_stores(g, 1, ssem_b)

            drain_gathers(0, gsem_a)
            drain_stores(1, ssem_b)
            fire_gathers(g + 2, 1, gsem_b)
            fire_stores(g + 1, 0, ssem_a)

        drain_gathers(1, gsem_b)
        drain_stores(0, ssem_a)
        fire_stores(n_groups - 1, 1, ssem_b)
        drain_stores(1, ssem_b)

    return emb_kernel


def kernel(x, W):
    B, H = x.shape
    V, D = W.shape
    n_flat = B * H
    assert n_flat % (NW * CHUNK) == 0 and D == 64
    n_chunks = n_flat // (NW * CHUNK)

    tr, vpad, rem = _build_transpose(V, D)
    assert rem > 0 and rem % 2 == 0
    wtail = W[V - rem:].reshape(rem // 2, 2 * D)
    Wd = tr(W.T, wtail)
    table = Wd.reshape(vpad, D)

    idx = x.reshape(NW, n_chunks, CHUNK)
    out = _build_gather(n_chunks, D)(idx, table)
    return out.reshape(B, H * D)


# speedup vs baseline: 1.4764x; 1.0063x over previous
import functools

import jax
import jax.numpy as jnp
from jax import lax
from jax.experimental import pallas as pl
from jax.experimental.pallas import tpu as pltpu
from jax.experimental.pallas import tpu_sc as plsc

NC = 2
NS = 16
NW = NC * NS
CHUNK = 128
K = 5
LANES = 16


@functools.cache
def _build_transpose(V, D):
    n_full = V // 128
    rem = V - n_full * 128
    vpad = (n_full + (1 if rem else 0)) * 128
    mesh = plsc.VectorSubcoreMesh(core_axis_name="c", subcore_axis_name="s")

    n_iter = (n_full - 1) // NW + 1
    if n_iter % 2:
        n_iter += 1

    @functools.partial(
        pl.kernel,
        out_type=jax.ShapeDtypeStruct((vpad // 2, 128), jnp.float32),
        mesh=mesh,
        scratch_types=[
            pltpu.VMEM((2, D, 128), jnp.float32),
            pltpu.VMEM((2, 64, 128), jnp.float32),
            pltpu.SemaphoreType.DMA,
            pltpu.SemaphoreType.DMA,
            pltpu.SemaphoreType.DMA,
            pltpu.SemaphoreType.DMA,
        ],
        compiler_params=pltpu.CompilerParams(needs_layout_passes=False),
    )
    def tr_kernel(wt_hbm, wtail_hbm, wd_hbm, tbuf, obuf,
                  isem_a, isem_b, osem_a, osem_b):
        wid = lax.axis_index("s") * NC + lax.axis_index("c")
        isems = (isem_a, isem_b)
        osems = (osem_a, osem_b)
        lane = jnp.arange(16, dtype=jnp.int32)
        rows = [lane // 2 + 8 * j for j in range(8)]
        colbase = (lane % 2) * 64

        def tile_of(k):
            return wid + NW * k

        def fire_in(k, par):
            @pl.when(tile_of(k) < n_full)
            def _():
                pltpu.async_copy(
                    wt_hbm.at[:, pl.ds(128 * tile_of(k), 128)],
                    tbuf.at[par], isems[par])

        def wait_in(par):
            pltpu.make_async_copy(
                wt_hbm.at[:, pl.ds(0, 128)], tbuf.at[par], isems[par]).wait()

        def fire_out(k, par):
            pltpu.async_copy(
                obuf.at[par], wd_hbm.at[pl.ds(64 * tile_of(k), 64), :],
                osems[par])

        def wait_out(par):
            pltpu.make_async_copy(
                obuf.at[par], wd_hbm.at[pl.ds(0, 64), :], osems[par]).wait()

        def compute(par):
            @pl.loop(0, D)
            def _(d):
                col = colbase + d
                vs = [tbuf[par, d, pl.ds(16 * j, 16)] for j in range(8)]
                for j in range(8):
                    plsc.store_scatter(obuf.at[par], [rows[j], col], vs[j])

        def step(k, par):
            fire_in(k + 1, 1 - par)

            @pl.when(tile_of(k) < n_full)
            def _():
                wait_in(par)

            @pl.when(jnp.logical_and(k >= 2, tile_of(k - 2) < n_full))
            def _():
                wait_out(par)

            @pl.when(tile_of(k) < n_full)
            def _():
                compute(par)
                fire_out(k, par)

        fire_in(0, 0)

        @pl.loop(0, n_iter, step=2)
        def _(k):
            step(k, 0)
            step(k + 1, 1)

        for m in (n_iter - 2, n_iter - 1):
            @pl.when(tile_of(m) < n_full)
            def _():
                wait_out(m % 2)

        if rem:
            @pl.when(wid == NW - 1)
            def _():
                pltpu.sync_copy(wtail_hbm, obuf.at[0, pl.ds(0, rem // 2), :])
                pltpu.sync_copy(obuf.at[0, pl.ds(0, rem // 2), :],
                                wd_hbm.at[pl.ds(64 * n_full, rem // 2), :])

    return tr_kernel, vpad, rem


@functools.cache
def _build_gather(n_chunks, D):
    n_groups = n_chunks // K
    assert n_groups % 2 == 0 and n_groups >= 4
    mesh = plsc.VectorSubcoreMesh(core_axis_name="c", subcore_axis_name="s")

    @functools.partial(
        pl.kernel,
        out_type=jax.ShapeDtypeStruct((NW, n_chunks, CHUNK, D), jnp.float32),
        mesh=mesh,
        scratch_types=[
            pltpu.VMEM((n_chunks, CHUNK), jnp.int32),
            pltpu.VMEM((2, K, CHUNK, D), jnp.float32),
            pltpu.SemaphoreType.DMA,
            pltpu.SemaphoreType.DMA,
            pltpu.SemaphoreType.DMA,
            pltpu.SemaphoreType.DMA,
        ],
        compiler_params=pltpu.CompilerParams(use_tc_tiling_on_sc=False),
    )
    def emb_kernel(idx_hbm, table_hbm, out_hbm, idx_v, rows_v,
                   gsem_a, gsem_b, ssem_a, ssem_b):
        wid = lax.axis_index("s") * NC + lax.axis_index("c")
        pltpu.sync_copy(idx_hbm.at[wid], idx_v)

        def fire_gathers(g, half, sem):
            for b in range(K):
                pltpu.async_copy(
                    table_hbm.at[idx_v.at[g * K + b]], rows_v.at[half, b], sem)

        def drain_gathers(half, sem):
            for b in range(K):
                pltpu.make_async_copy(
                    table_hbm.at[idx_v.at[0]], rows_v.at[half, b], sem).wait()

        def fire_stores(g, half, sem):
            for b in range(K):
                pltpu.async_copy(
                    rows_v.at[half, b], out_hbm.at[wid, g * K + b], sem)

        def drain_stores(half, sem):
            for b in range(K):
                pltpu.make_async_copy(
                    rows_v.at[half, b], out_hbm.at[wid, b], sem).wait()

        fire_gathers(0, 0, gsem_a)
        drain_gathers(0, gsem_a)
        fire_gathers(1, 1, gsem_b)
        fire_stores(0, 0, ssem_a)

        @pl.loop(1, n_groups - 1, step=2)
        def _(g):
            drain_gathers(1, gsem_b)
            drain_stores(0, ssem_a)
            fire_gathers(g + 1, 0, gsem_a)
            fire_stores(g, 1, ssem_b)

            drain_gathers(0, gsem_a)
            drain_stores(1, ssem_b)
            fire_gathers(g + 2, 1, gsem_b)
            fire_stores(g + 1, 0, ssem_a)

        drain_gathers(1, gsem_b)
        drain_stores(0, ssem_a)
        fire_stores(n_groups - 1, 1, ssem_b)
        drain_stores(1, ssem_b)

    return emb_kernel


def kernel(x, W):
    B, H = x.shape
    V, D = W.shape
    n_flat = B * H
    assert n_flat % (NW * CHUNK) == 0 and D == 64
    n_chunks = n_flat // (NW * CHUNK)

    tr, vpad, rem = _build_transpose(V, D)
    assert rem > 0 and rem % 2 == 0
    wtail = W[V - rem:].reshape(rem // 2, 2 * D)
    Wd = tr(W.T, wtail)
    table = Wd.reshape(vpad, D)

    idx = x.reshape(NW, n_chunks, CHUNK)
    out = _build_gather(n_chunks, D)(idx, table)
    return out.reshape(B, H * D)
